# Initial kernel scaffold; baseline (speedup 1.0000x reference)
import jax
import jax.numpy as jnp
from jax.experimental import pallas as pl


def kernel(q, local_out, mem_k, mem_v, g):
    raise NotImplementedError("write your pallas kernel here")


# speedup vs baseline: 24.3203x; 24.3203x over previous
import functools

import jax
import jax.numpy as jnp
from jax import lax
from jax.experimental import pallas as pl
from jax.experimental.pallas import tpu as pltpu
from jax.experimental.pallas import tpu_sc as plsc

_NEG_INF = float("-inf")

_NC = 2
_NS = 16
_L = 16
_NW = _NC * _NS


def _topk_body(nm, mb, g_ref, q_ref, k_ref, attn_ref, idx_ref, bv_ref, bi_ref):
    j = pl.program_id(0)
    s_rows = q_ref.shape[0]

    @pl.when(j == 0)
    def _():
        bv_ref[...] = jnp.full(bv_ref.shape, _NEG_INF, dtype=jnp.float32)
        bi_ref[...] = jnp.zeros(bi_ref.shape, dtype=jnp.int32)

    s = lax.dot_general(
        q_ref[...], k_ref[...], (((1,), (1,)), ((), ())),
        preferred_element_type=jnp.float32,
    )

    iota = lax.broadcasted_iota(jnp.int32, (s_rows, mb), 1)
    iota8 = lax.broadcasted_iota(jnp.int32, (s_rows, 8), 1)

    av = bv_ref[...]
    ai = bi_ref[...]
    for r in range(4):
        m = jnp.max(s, axis=1, keepdims=True)
        p = jnp.min(jnp.where(s == m, iota, mb), axis=1, keepdims=True)
        s = jnp.where(iota == p, _NEG_INF, s)
        av = jnp.where(iota8 == 4 + r, jnp.broadcast_to(m, av.shape), av)
        ai = jnp.where(iota8 == 4 + r,
                       jnp.broadcast_to(p + j * mb, ai.shape), ai)

    work = av
    out_v = av
    out_i = ai
    for r in range(4):
        m = jnp.max(work, axis=1, keepdims=True)
        p8 = jnp.min(jnp.where(work == m, iota8, 8), axis=1, keepdims=True)
        iv = jnp.sum(jnp.where(iota8 == p8, ai, 0), axis=1, keepdims=True)
        work = jnp.where(iota8 == p8, _NEG_INF, work)
        out_v = jnp.where(iota8 == r, jnp.broadcast_to(m, out_v.shape), out_v)
        out_i = jnp.where(iota8 == r, jnp.broadcast_to(iv, out_i.shape), out_i)
    bv_ref[...] = out_v
    bi_ref[...] = out_i

    @pl.when(j == nm - 1)
    def _():
        d = q_ref.shape[1]
        scale = 1.0 / (jnp.float32(d) ** 0.5)
        tv = out_v[:, 0:4] * scale
        mx = jnp.max(tv, axis=1, keepdims=True)
        e = jnp.exp(tv - mx)
        w = e / jnp.sum(e, axis=1, keepdims=True)
        attn_ref[...] = w * g_ref[...]
        idx_ref[...] = out_i[:, 0:4]


def _topk_attn(qs, mk, g):
    s_rows, d = qs.shape
    m_rows = mk.shape[0]
    mb = min(512, m_rows)
    nm = m_rows // mb
    return pl.pallas_call(
        functools.partial(_topk_body, nm, mb),
        grid=(nm,),
        in_specs=[
            pl.BlockSpec((1, 1), lambda j: (0, 0)),
            pl.BlockSpec((s_rows, d), lambda j: (0, 0)),
            pl.BlockSpec((mb, d), lambda j: (j, 0)),
        ],
        out_specs=[
            pl.BlockSpec((s_rows, 4), lambda j: (0, 0)),
            pl.BlockSpec((s_rows, 4), lambda j: (0, 0)),
        ],
        out_shape=[
            jax.ShapeDtypeStruct((s_rows, 4), jnp.float32),
            jax.ShapeDtypeStruct((s_rows, 4), jnp.int32),
        ],
        scratch_shapes=[
            pltpu.VMEM((s_rows, 8), jnp.float32),
            pltpu.VMEM((s_rows, 8), jnp.int32),
        ],
        compiler_params=pltpu.CompilerParams(
            dimension_semantics=("arbitrary",),
        ),
    )(g.reshape(1, 1), qs, mk)


def _sc_combine(mv, idx_flat, attn_exp, lo):
    s_rows, d = lo.shape
    rows_per_w = s_rows // _NW
    ch = 16
    n_chunks = rows_per_w // ch
    mesh = plsc.VectorSubcoreMesh(core_axis_name="c", subcore_axis_name="s")

    @functools.partial(
        pl.kernel,
        mesh=mesh,
        out_type=jax.ShapeDtypeStruct((s_rows, d), jnp.float32),
        scratch_types=[
            pltpu.VMEM((ch * 4,), jnp.int32),
            pltpu.VMEM((ch * 4, d), jnp.float32),
            pltpu.VMEM((ch * 4, _L), jnp.float32),
            pltpu.VMEM((ch, d), jnp.float32),
            pltpu.VMEM((ch, d), jnp.float32),
            pltpu.SemaphoreType.DMA,
        ],
    )
    def k(mv_hbm, idx_hbm, attn_hbm, lo_hbm, out_hbm,
          idx_v, rows_v, attn_v, lo_v, out_v, sem):
        wid = lax.axis_index("s") * _NC + lax.axis_index("c")
        base = wid * rows_per_w
        for c in range(n_chunks):
            qbase = base + c * ch
            ibase = qbase * 4
            pltpu.sync_copy(idx_hbm.at[pl.ds(ibase, ch * 4)], idx_v)
            pltpu.async_copy(mv_hbm.at[idx_v], rows_v, sem).wait()
            pltpu.sync_copy(attn_hbm.at[pl.ds(ibase, ch * 4)], attn_v)
            pltpu.sync_copy(lo_hbm.at[pl.ds(qbase, ch)], lo_v)

            @pl.loop(0, ch)
            def _(w):
                wv0 = attn_v.at[pl.ds(4 * w + 0, 1), :][...]
                wv1 = attn_v.at[pl.ds(4 * w + 1, 1), :][...]
                wv2 = attn_v.at[pl.ds(4 * w + 2, 1), :][...]
                wv3 = attn_v.at[pl.ds(4 * w + 3, 1), :][...]

                @pl.loop(0, d, step=_L)
                def _(col):
                    sl = pl.ds(col, _L)
                    acc = lo_v.at[pl.ds(w, 1), sl][...]
                    acc = acc + wv0 * rows_v.at[pl.ds(4 * w + 0, 1), sl][...]
                    acc = acc + wv1 * rows_v.at[pl.ds(4 * w + 1, 1), sl][...]
                    acc = acc + wv2 * rows_v.at[pl.ds(4 * w + 2, 1), sl][...]
                    acc = acc + wv3 * rows_v.at[pl.ds(4 * w + 3, 1), sl][...]
                    out_v.at[pl.ds(w, 1), sl][...] = acc

            pltpu.sync_copy(out_v, out_hbm.at[pl.ds(qbase, ch)])

    return k(mv, idx_flat, attn_exp, lo)


def kernel(q, local_out, mem_k, mem_v, g):
    b, s_rows, d = q.shape
    qs = q.reshape(s_rows, d)
    mk = mem_k.reshape(-1, d)
    mv = mem_v.reshape(-1, d)
    lo = local_out.reshape(s_rows, d)

    attn, idx = _topk_attn(qs, mk, g)
    attn_exp = jnp.broadcast_to(attn.reshape(s_rows * 4, 1), (s_rows * 4, _L))
    idx_flat = idx.reshape(s_rows * 4)
    out = _sc_combine(mv, idx_flat, attn_exp, lo)
    return out.reshape(b, s_rows, d)


# speedup vs baseline: 45.0920x; 1.8541x over previous
import functools

import jax
import jax.numpy as jnp
from jax import lax
from jax.experimental import pallas as pl
from jax.experimental.pallas import tpu as pltpu
from jax.experimental.pallas import tpu_sc as plsc

_NEG_INF = float("-inf")

_NC = 2
_NS = 16
_L = 16
_NW = _NC * _NS


def _topk_body(nm, mb, g_ref, q_ref, k_ref, attn_ref, idx_ref, cv_ref, ci_ref):
    j = pl.program_id(0)
    s_rows = q_ref.shape[0]
    ncand = 4 * nm

    s = lax.dot_general(
        q_ref[...], k_ref[...], (((1,), (1,)), ((), ())),
        preferred_element_type=jnp.float32,
    )

    iota_f = lax.broadcasted_iota(jnp.int32, (s_rows, mb), 1).astype(jnp.float32)
    mbf = jnp.float32(mb)
    jbase = (j * mb).astype(jnp.float32)

    iota_c = lax.broadcasted_iota(jnp.int32, (s_rows, ncand), 1)
    cv = cv_ref[...]
    ci = ci_ref[...]
    for r in range(4):
        m = jnp.max(s, axis=1, keepdims=True)
        eq = s == m
        pf = jnp.min(jnp.where(eq, iota_f, mbf), axis=1, keepdims=True)
        if r < 3:
            s = jnp.where(iota_f == pf, _NEG_INF, s)
        col = iota_c == 4 * j + r
        cv = jnp.where(col, jnp.broadcast_to(m, cv.shape), cv)
        ci = jnp.where(col, jnp.broadcast_to(pf + jbase, ci.shape), ci)
    cv_ref[...] = cv
    ci_ref[...] = ci

    @pl.when(j == nm - 1)
    def _():
        cv = cv_ref[...]
        ci = ci_ref[...]
        iota_cf = lax.broadcasted_iota(
            jnp.int32, (s_rows, ncand), 1).astype(jnp.float32)
        iota4 = lax.broadcasted_iota(jnp.int32, (s_rows, 4), 1)
        tv = jnp.zeros((s_rows, 4), jnp.float32)
        ti = jnp.zeros((s_rows, 4), jnp.float32)
        for r in range(4):
            m = jnp.max(cv, axis=1, keepdims=True)
            pf = jnp.min(jnp.where(cv == m, iota_cf, float(ncand)),
                         axis=1, keepdims=True)
            sel = iota_cf == pf
            iv = jnp.sum(jnp.where(sel, ci, 0.0), axis=1, keepdims=True)
            if r < 3:
                cv = jnp.where(sel, _NEG_INF, cv)
            tv = jnp.where(iota4 == r, jnp.broadcast_to(m, tv.shape), tv)
            ti = jnp.where(iota4 == r, jnp.broadcast_to(iv, ti.shape), ti)
        d = q_ref.shape[1]
        scale = 1.0 / (jnp.float32(d) ** 0.5)
        tvs = tv * scale
        mx = jnp.max(tvs, axis=1, keepdims=True)
        e = jnp.exp(tvs - mx)
        w = e / jnp.sum(e, axis=1, keepdims=True)
        attn_ref[...] = w * g_ref[...]
        idx_ref[...] = ti.astype(jnp.int32)


def _topk_attn(qs, mk, g):
    s_rows, d = qs.shape
    m_rows = mk.shape[0]
    mb = min(512, m_rows)
    nm = m_rows // mb
    return pl.pallas_call(
        functools.partial(_topk_body, nm, mb),
        grid=(nm,),
        in_specs=[
            pl.BlockSpec((1, 1), lambda j: (0, 0)),
            pl.BlockSpec((s_rows, d), lambda j: (0, 0)),
            pl.BlockSpec((mb, d), lambda j: (j, 0)),
        ],
        out_specs=[
            pl.BlockSpec((s_rows, 4), lambda j: (0, 0)),
            pl.BlockSpec((s_rows, 4), lambda j: (0, 0)),
        ],
        out_shape=[
            jax.ShapeDtypeStruct((s_rows, 4), jnp.float32),
            jax.ShapeDtypeStruct((s_rows, 4), jnp.int32),
        ],
        scratch_shapes=[
            pltpu.VMEM((s_rows, 4 * nm), jnp.float32),
            pltpu.VMEM((s_rows, 4 * nm), jnp.float32),
        ],
        compiler_params=pltpu.CompilerParams(
            dimension_semantics=("arbitrary",),
        ),
    )(g.reshape(1, 1), qs, mk)


def _sc_combine(mv, idx_flat, attn_exp, lo):
    s_rows, d = lo.shape
    rows_per_w = s_rows // _NW
    ch = 16
    n_chunks = rows_per_w // ch
    mesh = plsc.VectorSubcoreMesh(core_axis_name="c", subcore_axis_name="s")

    @functools.partial(
        pl.kernel,
        mesh=mesh,
        out_type=jax.ShapeDtypeStruct((s_rows, d), jnp.float32),
        scratch_types=[
            pltpu.VMEM((ch * 4,), jnp.int32),
            pltpu.VMEM((ch * 4, d), jnp.float32),
            pltpu.VMEM((ch * 4, _L), jnp.float32),
            pltpu.VMEM((ch, d), jnp.float32),
            pltpu.VMEM((ch, d), jnp.float32),
            pltpu.SemaphoreType.DMA,
        ],
    )
    def k(mv_hbm, idx_hbm, attn_hbm, lo_hbm, out_hbm,
          idx_v, rows_v, attn_v, lo_v, out_v, sem):
        wid = lax.axis_index("s") * _NC + lax.axis_index("c")
        base = wid * rows_per_w
        for c in range(n_chunks):
            qbase = base + c * ch
            ibase = qbase * 4
            pltpu.sync_copy(idx_hbm.at[pl.ds(ibase, ch * 4)], idx_v)
            pltpu.async_copy(mv_hbm.at[idx_v], rows_v, sem).wait()
            pltpu.sync_copy(attn_hbm.at[pl.ds(ibase, ch * 4)], attn_v)
            pltpu.sync_copy(lo_hbm.at[pl.ds(qbase, ch)], lo_v)

            @pl.loop(0, ch)
            def _(w):
                wv0 = attn_v.at[pl.ds(4 * w + 0, 1), :][...]
                wv1 = attn_v.at[pl.ds(4 * w + 1, 1), :][...]
                wv2 = attn_v.at[pl.ds(4 * w + 2, 1), :][...]
                wv3 = attn_v.at[pl.ds(4 * w + 3, 1), :][...]

                @pl.loop(0, d, step=_L)
                def _(col):
                    sl = pl.ds(col, _L)
                    acc = lo_v.at[pl.ds(w, 1), sl][...]
                    acc = acc + wv0 * rows_v.at[pl.ds(4 * w + 0, 1), sl][...]
                    acc = acc + wv1 * rows_v.at[pl.ds(4 * w + 1, 1), sl][...]
                    acc = acc + wv2 * rows_v.at[pl.ds(4 * w + 2, 1), sl][...]
                    acc = acc + wv3 * rows_v.at[pl.ds(4 * w + 3, 1), sl][...]
                    out_v.at[pl.ds(w, 1), sl][...] = acc

            pltpu.sync_copy(out_v, out_hbm.at[pl.ds(qbase, ch)])

    return k(mv, idx_flat, attn_exp, lo)


def kernel(q, local_out, mem_k, mem_v, g):
    b, s_rows, d = q.shape
    qs = q.reshape(s_rows, d)
    mk = mem_k.reshape(-1, d)
    mv = mem_v.reshape(-1, d)
    lo = local_out.reshape(s_rows, d)

    attn, idx = _topk_attn(qs, mk, g)
    attn_exp = jnp.broadcast_to(attn.reshape(s_rows * 4, 1), (s_rows * 4, _L))
    idx_flat = idx.reshape(s_rows * 4)
    out = _sc_combine(mv, idx_flat, attn_exp, lo)
    return out.reshape(b, s_rows, d)


# speedup vs baseline: 46.7925x; 1.0377x over previous
import functools

import jax
import jax.numpy as jnp
from jax import lax
from jax.experimental import pallas as pl
from jax.experimental.pallas import tpu as pltpu
from jax.experimental.pallas import tpu_sc as plsc

_NEG_INF = float("-inf")

_NC = 2
_NS = 16
_L = 16
_NW = _NC * _NS


def _topk_body(nm, mb, g_ref, q_ref, k_ref, attn_ref, idx_ref, cv_ref, ci_ref):
    j = pl.program_id(0)
    s_rows = q_ref.shape[0]
    ncand = 4 * nm

    s = lax.dot_general(
        q_ref[...], k_ref[...], (((1,), (1,)), ((), ())),
        preferred_element_type=jnp.float32,
    )

    iota_f = lax.broadcasted_iota(jnp.int32, (s_rows, mb), 1).astype(jnp.float32)
    mbf = jnp.float32(mb)
    jbase = (j * mb).astype(jnp.float32)

    iota_c = lax.broadcasted_iota(jnp.int32, (s_rows, ncand), 1)
    cv = cv_ref[...]
    ci = ci_ref[...]
    for r in range(4):
        m = jnp.max(s, axis=1, keepdims=True)
        eq = s == m
        pf = jnp.min(jnp.where(eq, iota_f, mbf), axis=1, keepdims=True)
        if r < 3:
            s = jnp.where(iota_f == pf, _NEG_INF, s)
        col = iota_c == 4 * j + r
        cv = jnp.where(col, jnp.broadcast_to(m, cv.shape), cv)
        ci = jnp.where(col, jnp.broadcast_to(pf + jbase, ci.shape), ci)
    cv_ref[...] = cv
    ci_ref[...] = ci

    @pl.when(j == nm - 1)
    def _():
        cv = cv_ref[...]
        ci = ci_ref[...]
        iota_cf = lax.broadcasted_iota(
            jnp.int32, (s_rows, ncand), 1).astype(jnp.float32)
        iota4 = lax.broadcasted_iota(jnp.int32, (s_rows, 4), 1)
        tv = jnp.zeros((s_rows, 4), jnp.float32)
        ti = jnp.zeros((s_rows, 4), jnp.float32)
        for r in range(4):
            m = jnp.max(cv, axis=1, keepdims=True)
            pf = jnp.min(jnp.where(cv == m, iota_cf, float(ncand)),
                         axis=1, keepdims=True)
            sel = iota_cf == pf
            iv = jnp.sum(jnp.where(sel, ci, 0.0), axis=1, keepdims=True)
            if r < 3:
                cv = jnp.where(sel, _NEG_INF, cv)
            tv = jnp.where(iota4 == r, jnp.broadcast_to(m, tv.shape), tv)
            ti = jnp.where(iota4 == r, jnp.broadcast_to(iv, ti.shape), ti)
        d = q_ref.shape[1]
        scale = 1.0 / (jnp.float32(d) ** 0.5)
        tvs = tv * scale
        mx = jnp.max(tvs, axis=1, keepdims=True)
        e = jnp.exp(tvs - mx)
        w = e / jnp.sum(e, axis=1, keepdims=True)
        attn_ref[...] = w * g_ref[...]
        idx_ref[...] = ti.astype(jnp.int32)


def _topk_attn(qs, mk, g):
    s_rows, d = qs.shape
    m_rows = mk.shape[0]
    mb = min(2048, m_rows)
    nm = m_rows // mb
    return pl.pallas_call(
        functools.partial(_topk_body, nm, mb),
        grid=(nm,),
        in_specs=[
            pl.BlockSpec((1, 1), lambda j: (0, 0)),
            pl.BlockSpec((s_rows, d), lambda j: (0, 0)),
            pl.BlockSpec((mb, d), lambda j: (j, 0)),
        ],
        out_specs=[
            pl.BlockSpec((s_rows, 4), lambda j: (0, 0)),
            pl.BlockSpec((s_rows, 4), lambda j: (0, 0)),
        ],
        out_shape=[
            jax.ShapeDtypeStruct((s_rows, 4), jnp.float32),
            jax.ShapeDtypeStruct((s_rows, 4), jnp.int32),
        ],
        scratch_shapes=[
            pltpu.VMEM((s_rows, 4 * nm), jnp.float32),
            pltpu.VMEM((s_rows, 4 * nm), jnp.float32),
        ],
        compiler_params=pltpu.CompilerParams(
            dimension_semantics=("arbitrary",),
        ),
    )(g.reshape(1, 1), qs, mk)


def _sc_combine(mv, idx_flat, attn_exp, lo):
    s_rows, d = lo.shape
    rows_per_w = s_rows // _NW
    ch = 16
    n_chunks = rows_per_w // ch
    mesh = plsc.VectorSubcoreMesh(core_axis_name="c", subcore_axis_name="s")

    @functools.partial(
        pl.kernel,
        mesh=mesh,
        out_type=jax.ShapeDtypeStruct((s_rows, d), jnp.float32),
        scratch_types=[
            pltpu.VMEM((ch * 4,), jnp.int32),
            pltpu.VMEM((ch * 4, d), jnp.float32),
            pltpu.VMEM((ch * 4, _L), jnp.float32),
            pltpu.VMEM((ch, d), jnp.float32),
            pltpu.VMEM((ch, d), jnp.float32),
            pltpu.SemaphoreType.DMA,
        ],
    )
    def k(mv_hbm, idx_hbm, attn_hbm, lo_hbm, out_hbm,
          idx_v, rows_v, attn_v, lo_v, out_v, sem):
        wid = lax.axis_index("s") * _NC + lax.axis_index("c")
        base = wid * rows_per_w
        for c in range(n_chunks):
            qbase = base + c * ch
            ibase = qbase * 4
            pltpu.sync_copy(idx_hbm.at[pl.ds(ibase, ch * 4)], idx_v)
            pltpu.async_copy(mv_hbm.at[idx_v], rows_v, sem).wait()
            pltpu.sync_copy(attn_hbm.at[pl.ds(ibase, ch * 4)], attn_v)
            pltpu.sync_copy(lo_hbm.at[pl.ds(qbase, ch)], lo_v)

            @pl.loop(0, ch)
            def _(w):
                wv0 = attn_v.at[pl.ds(4 * w + 0, 1), :][...]
                wv1 = attn_v.at[pl.ds(4 * w + 1, 1), :][...]
                wv2 = attn_v.at[pl.ds(4 * w + 2, 1), :][...]
                wv3 = attn_v.at[pl.ds(4 * w + 3, 1), :][...]

                @pl.loop(0, d, step=_L)
                def _(col):
                    sl = pl.ds(col, _L)
                    acc = lo_v.at[pl.ds(w, 1), sl][...]
                    acc = acc + wv0 * rows_v.at[pl.ds(4 * w + 0, 1), sl][...]
                    acc = acc + wv1 * rows_v.at[pl.ds(4 * w + 1, 1), sl][...]
                    acc = acc + wv2 * rows_v.at[pl.ds(4 * w + 2, 1), sl][...]
                    acc = acc + wv3 * rows_v.at[pl.ds(4 * w + 3, 1), sl][...]
                    out_v.at[pl.ds(w, 1), sl][...] = acc

            pltpu.sync_copy(out_v, out_hbm.at[pl.ds(qbase, ch)])

    return k(mv, idx_flat, attn_exp, lo)


def kernel(q, local_out, mem_k, mem_v, g):
    b, s_rows, d = q.shape
    qs = q.reshape(s_rows, d)
    mk = mem_k.reshape(-1, d)
    mv = mem_v.reshape(-1, d)
    lo = local_out.reshape(s_rows, d)

    attn, idx = _topk_attn(qs, mk, g)
    attn_exp = jnp.broadcast_to(attn.reshape(s_rows * 4, 1), (s_rows * 4, _L))
    idx_flat = idx.reshape(s_rows * 4)
    out = _sc_combine(mv, idx_flat, attn_exp, lo)
    return out.reshape(b, s_rows, d)
